# baseline (speedup 1.0000x reference)
import functools

import jax
import jax.numpy as jnp
from jax import lax
from jax.experimental import pallas as pl
from jax.experimental.pallas import tpu as pltpu
from jax.experimental.pallas import tpu_sc as plsc

N = 10000
E = 160000
D = 256
EPS = 1e-5

NPH = 4
DQ = D // NPH

NC = 2
NS = 16
B = 128
NCH = 80
EROWS = NS * NCH
EPAD = EROWS * B - E
NBUF = 4
CAP = NCH * B + NBUF * B
NHALF = N // NC
GROW = NHALF
ACCR = NHALF + 8
SLAB = 312
TAIL = NHALF - NS * SLAB
TAIL_OFF = NS * SLAB

RB = 1000
NRB = N // RB


def _ln_body(x_ref, g_ref, b_ref, o_ref):
    xb = x_ref[...]
    mu = jnp.mean(xb, axis=-1, keepdims=True)
    xc = xb - mu
    var = jnp.mean(xc * xc, axis=-1, keepdims=True)
    xn = xc * lax.rsqrt(var + EPS) * g_ref[...] + b_ref[...]
    h = jnp.maximum(xn, 0.0)
    for q in range(NPH):
        o_ref[q] = h[:, q * DQ:(q + 1) * DQ]


_ln_call = pl.pallas_call(
    _ln_body,
    grid=(NRB,),
    in_specs=[
        pl.BlockSpec((RB, D), lambda i: (i, 0)),
        pl.BlockSpec((1, D), lambda i: (0, 0)),
        pl.BlockSpec((1, D), lambda i: (0, 0)),
    ],
    out_specs=pl.BlockSpec((NPH, RB, DQ), lambda i: (0, i, 0)),
    out_shape=jax.ShapeDtypeStruct((NPH, N, DQ), jnp.float32),
)


_mesh = plsc.VectorSubcoreMesh(
    core_axis_name="c", subcore_axis_name="s", num_cores=NC, num_subcores=NS
)


@functools.partial(
    pl.kernel,
    out_type=(
        jax.ShapeDtypeStruct((NPH * N, DQ), jnp.float32),
        jax.ShapeDtypeStruct((N, 16), jnp.float32),
    ),
    mesh=_mesh,
    compiler_params=pltpu.CompilerParams(use_tc_tiling_on_sc=False,
                                         needs_layout_passes=False),
    scratch_types=(
        pltpu.VMEM((NCH, B), jnp.int32),
        pltpu.VMEM((NCH, B), jnp.int32),
        pltpu.VMEM((CAP,), jnp.int32),
        pltpu.VMEM((CAP,), jnp.int32),
        pltpu.VMEM((16,), jnp.int32),
        pltpu.VMEM((NBUF, B, DQ), jnp.float32),
        pltpu.VMEM((B, 16), jnp.float32),
        pltpu.VMEM((SLAB, DQ), jnp.float32),
        pltpu.VMEM((SLAB, 16), jnp.float32),
        pltpu.VMEM_SHARED((ACCR, DQ), jnp.float32),
        pltpu.VMEM_SHARED((ACCR, 16), jnp.float32),
        pltpu.SemaphoreType.DMA,
        pltpu.SemaphoreType.DMA,
        pltpu.SemaphoreType.DMA,
        pltpu.SemaphoreType.DMA,
        pltpu.SemaphoreType.DMA,
        pltpu.SemaphoreType.DMA,
        pltpu.SemaphoreType.DMA,
        pltpu.SemaphoreType.DMA,
        pltpu.SemaphoreType.DMA,
    ),
)
def _sc_aggregate(src_hbm, dst_hbm, hq_hbm, zrows_hbm, zcnt_hbm, ones_hbm,
                  agg_out, cnt_out,
                  srcr_t, dstr_t, srcc_t, dstc_t, mbuf_v, rows_v, ones_v,
                  stage_v, stagec_v, acc_sh, cnt_sh,
                  sem_g0, sem_g1, sem_g2, sem_g3,
                  sem_s0, sem_s1, sem_s2, sem_s3, sem_c):
    c = lax.axis_index("c")
    s = lax.axis_index("s")
    cbase = c * NHALF
    sem_g = (sem_g0, sem_g1, sem_g2, sem_g3)
    sem_s = (sem_s0, sem_s1, sem_s2, sem_s3)

    pltpu.sync_copy(ones_hbm, ones_v)
    pltpu.sync_copy(src_hbm.at[pl.ds(s * NCH, NCH)], srcr_t)
    pltpu.sync_copy(dst_hbm.at[pl.ds(s * NCH, NCH)], dstr_t)

    def _compact_row(j, mvec):
        for i in range(B // 16):
            sl = pl.ds(i * 16, 16)
            t = dstr_t[j, sl] - cbase
            valid = jnp.logical_and(t >= 0, t < NHALF)
            pos = mvec - 1 + plsc.cumsum(jnp.where(valid, 1, 0))
            plsc.store_scatter(dstc_t, [pos], t, mask=valid)
            plsc.store_scatter(srcc_t, [pos], srcr_t[j, sl], mask=valid)
            mvec = mvec + plsc.all_reduce_population_count(valid)
        return mvec

    mvec = lax.fori_loop(0, NCH, _compact_row,
                         jnp.zeros((16,), jnp.int32))
    mbuf_v[...] = mvec
    m = mbuf_v[pl.ds(0, 16)][0]

    mpad = ((m + NBUF * B - 1) // (NBUF * B)) * (NBUF * B)
    zeros16 = jnp.zeros((16,), jnp.int32)
    grow16 = zeros16 + GROW

    def _pad(g, carry):
        idx = m + g * 16 + lax.iota(jnp.int32, 16)
        mask = idx < mpad
        plsc.store_scatter(dstc_t, [idx], grow16, mask=mask)
        plsc.store_scatter(srcc_t, [idx], zeros16, mask=mask)
        return carry

    lax.fori_loop(0, NBUF * B // 16, _pad, 0)
    nquad = mpad // (NBUF * B)
    nch = nquad * NBUF

    def _gather(j, d):
        return pltpu.async_copy(hq_hbm.at[srcc_t.at[pl.ds(j * B, B)]],
                                rows_v.at[d], sem_g[d])

    def _gather_wait(j, d):
        pltpu.make_async_copy(hq_hbm.at[srcc_t.at[pl.ds(j * B, B)]],
                              rows_v.at[d], sem_g[d]).wait()

    def _scat(j, d):
        pltpu.async_copy(rows_v.at[d], acc_sh.at[dstc_t.at[pl.ds(j * B, B)]],
                         sem_s[d], add=True)

    def _scat_wait(j, d):
        pltpu.make_async_copy(rows_v.at[d],
                              acc_sh.at[dstc_t.at[pl.ds(j * B, B)]],
                              sem_s[d]).wait()

    def _cnt(j):
        pltpu.async_copy(ones_v, cnt_sh.at[dstc_t.at[pl.ds(j * B, B)]],
                         sem_c, add=True)

    def _cnt_wait(j):
        pltpu.make_async_copy(ones_v, cnt_sh.at[dstc_t.at[pl.ds(j * B, B)]],
                              sem_c).wait()

    for p in range(NPH):
        if p > 0:
            def _bump(g, carry):
                sl = pl.ds(g * 16, 16)
                srcc_t[sl] = srcc_t[sl] + N
                return carry

            lax.fori_loop(0, CAP // 16, _bump, 0)

        pltpu.sync_copy(zrows_hbm, stage_v)
        pltpu.sync_copy(stage_v, acc_sh.at[pl.ds(s * SLAB, SLAB)])

        @pl.when(s == 0)
        def _():
            pltpu.sync_copy(stage_v.at[pl.ds(0, TAIL)],
                            acc_sh.at[pl.ds(TAIL_OFF, TAIL)])

        if p == 0:
            pltpu.sync_copy(zcnt_hbm, stagec_v)
            pltpu.sync_copy(stagec_v, cnt_sh.at[pl.ds(s * SLAB, SLAB)])

            @pl.when(s == 0)
            def _():
                pltpu.sync_copy(stagec_v.at[pl.ds(0, TAIL)],
                                cnt_sh.at[pl.ds(TAIL_OFF, TAIL)])

        plsc.subcore_barrier()

        @pl.when(nch > 0)
        def _():
            _gather(0, 0)

        @pl.when(nch > 1)
        def _():
            _gather(1, 1)

        def _pipe(k, carry):
            for u in range(NBUF):
                j = NBUF * k + u
                du = (u + 2) % NBUF

                @pl.when(j >= 2)
                def _():
                    _scat_wait(j - 2, du)

                @pl.when(j + 2 < nch)
                def _():
                    _gather(j + 2, du)

                _gather_wait(j, u)
                _scat(j, u)
                if p == 0:
                    @pl.when(j >= 2)
                    def _():
                        _cnt_wait(j - 2)

                    _cnt(j)
            return carry

        lax.fori_loop(0, nquad, _pipe, 0)

        @pl.when(nch > 0)
        def _():
            _scat_wait(nch - 2, NBUF - 2)
            _scat_wait(nch - 1, NBUF - 1)
            if p == 0:
                _cnt_wait(nch - 2)
                _cnt_wait(nch - 1)

        plsc.subcore_barrier()

        out0 = p * N + cbase
        pltpu.sync_copy(acc_sh.at[pl.ds(s * SLAB, SLAB)], stage_v)
        pltpu.sync_copy(stage_v, agg_out.at[pl.ds(out0 + s * SLAB, SLAB)])

        @pl.when(s == 0)
        def _():
            pltpu.sync_copy(acc_sh.at[pl.ds(TAIL_OFF, TAIL)],
                            stage_v.at[pl.ds(0, TAIL)])
            pltpu.sync_copy(stage_v.at[pl.ds(0, TAIL)],
                            agg_out.at[pl.ds(out0 + TAIL_OFF, TAIL)])

        if p == 0:
            pltpu.sync_copy(cnt_sh.at[pl.ds(s * SLAB, SLAB)], stagec_v)
            pltpu.sync_copy(stagec_v, cnt_out.at[pl.ds(cbase + s * SLAB, SLAB)])

            @pl.when(s == 0)
            def _():
                pltpu.sync_copy(cnt_sh.at[pl.ds(TAIL_OFF, TAIL)],
                                stagec_v.at[pl.ds(0, TAIL)])
                pltpu.sync_copy(stagec_v.at[pl.ds(0, TAIL)],
                                cnt_out.at[pl.ds(cbase + TAIL_OFF, TAIL)])


def _out_body(*refs):
    agg_refs = refs[:NPH]
    c_ref = refs[NPH]
    h_refs = refs[NPH + 1:2 * NPH + 1]
    wl_ref, bl_ref, wr_ref, o_ref = refs[2 * NPH + 1:]
    inv = 1.0 / jnp.maximum(c_ref[:, 0:1], 1.0)
    dn = (((1,), (1,)), ((), ()))
    acc = bl_ref[...] + jnp.zeros((RB, D), jnp.float32)
    for q in range(NPH):
        acc += lax.dot_general(agg_refs[q][...] * inv,
                               wl_ref[:, q * DQ:(q + 1) * DQ], dn,
                               preferred_element_type=jnp.float32)
        acc += lax.dot_general(h_refs[q][...],
                               wr_ref[:, q * DQ:(q + 1) * DQ], dn,
                               preferred_element_type=jnp.float32)
    o_ref[...] = acc


def _group_spec(q):
    return pl.BlockSpec((RB, DQ), lambda i, q=q: (q * NRB + i, 0))


_out_call = pl.pallas_call(
    _out_body,
    grid=(NRB,),
    in_specs=(
        [_group_spec(q) for q in range(NPH)]
        + [pl.BlockSpec((RB, 16), lambda i: (i, 0))]
        + [_group_spec(q) for q in range(NPH)]
        + [
            pl.BlockSpec((D, D), lambda i: (0, 0)),
            pl.BlockSpec((1, D), lambda i: (0, 0)),
            pl.BlockSpec((D, D), lambda i: (0, 0)),
        ]
    ),
    out_specs=pl.BlockSpec((RB, D), lambda i: (i, 0)),
    out_shape=jax.ShapeDtypeStruct((N, D), jnp.float32),
)


def kernel(x, edge_index, gamma, beta, W_l, b_l, W_r):
    hq = _ln_call(x, gamma.reshape(1, D), beta.reshape(1, D))
    hqf = hq.reshape(NPH * N, DQ)
    src = jnp.concatenate(
        [edge_index[0], jnp.zeros((EPAD,), jnp.int32)]).reshape(EROWS, B)
    dst = jnp.concatenate(
        [edge_index[1], jnp.full((EPAD,), N, jnp.int32)]).reshape(EROWS, B)
    zrows = jnp.zeros((SLAB, DQ), jnp.float32)
    zcnt = jnp.zeros((SLAB, 16), jnp.float32)
    ones = jnp.ones((B, 16), jnp.float32)
    agg, cnt = _sc_aggregate(src, dst, hqf, zrows, zcnt, ones)
    args = ([agg] * NPH) + [cnt] + ([hqf] * NPH)
    return _out_call(*args, W_l, b_l.reshape(1, D), W_r)


# speedup vs baseline: 1.1695x; 1.1695x over previous
import functools

import jax
import jax.numpy as jnp
from jax import lax
from jax.experimental import pallas as pl
from jax.experimental.pallas import tpu as pltpu
from jax.experimental.pallas import tpu_sc as plsc

N = 10000
E = 160000
D = 256
EPS = 1e-5

NPH = 2
DQ = D // NPH

NC = 2
NS = 16
B = 128
NCH = 80
EROWS = NS * NCH
EPAD = EROWS * B - E
CAP = NCH * B + 4 * B

NQ = N // 4
GROW = NQ
ACCR = NQ + 12
SLAB = 156
TAIL = NQ - NS * SLAB
TAIL_OFF = NS * SLAB

RB = 1000
NRB = N // RB


def _ln_body(x_ref, g_ref, b_ref, o_ref):
    xb = x_ref[...]
    mu = jnp.mean(xb, axis=-1, keepdims=True)
    xc = xb - mu
    var = jnp.mean(xc * xc, axis=-1, keepdims=True)
    xn = xc * lax.rsqrt(var + EPS) * g_ref[...] + b_ref[...]
    h = jnp.maximum(xn, 0.0)
    for q in range(NPH):
        o_ref[q] = h[:, q * DQ:(q + 1) * DQ]


_ln_call = pl.pallas_call(
    _ln_body,
    grid=(NRB,),
    in_specs=[
        pl.BlockSpec((RB, D), lambda i: (i, 0)),
        pl.BlockSpec((1, D), lambda i: (0, 0)),
        pl.BlockSpec((1, D), lambda i: (0, 0)),
    ],
    out_specs=pl.BlockSpec((NPH, RB, DQ), lambda i: (0, i, 0)),
    out_shape=jax.ShapeDtypeStruct((NPH, N, DQ), jnp.float32),
)


_mesh = plsc.VectorSubcoreMesh(
    core_axis_name="c", subcore_axis_name="s", num_cores=NC, num_subcores=NS
)


@functools.partial(
    pl.kernel,
    out_type=(
        jax.ShapeDtypeStruct((NPH * N, DQ), jnp.float32),
        jax.ShapeDtypeStruct((N, 16), jnp.float32),
    ),
    mesh=_mesh,
    compiler_params=pltpu.CompilerParams(use_tc_tiling_on_sc=False,
                                         needs_layout_passes=False),
    scratch_types=(
        pltpu.VMEM((NCH, B), jnp.int32),
        pltpu.VMEM((NCH, B), jnp.int32),
        pltpu.VMEM((CAP,), jnp.int32),
        pltpu.VMEM((CAP,), jnp.int32),
        pltpu.VMEM((16,), jnp.int32),
        pltpu.VMEM((2, B, DQ), jnp.float32),
        pltpu.VMEM((B, 16), jnp.float32),
        pltpu.VMEM((SLAB, DQ), jnp.float32),
        pltpu.VMEM((SLAB, 16), jnp.float32),
        pltpu.VMEM_SHARED((ACCR, DQ), jnp.float32),
        pltpu.VMEM_SHARED((ACCR, 16), jnp.float32),
        pltpu.SemaphoreType.DMA,
        pltpu.SemaphoreType.DMA,
        pltpu.SemaphoreType.DMA,
        pltpu.SemaphoreType.DMA,
        pltpu.SemaphoreType.DMA,
    ),
)
def _sc_aggregate(src_hbm, dst_hbm, hq_hbm, zrows_hbm, zcnt_hbm, ones_hbm,
                  agg_out, cnt_out,
                  srcr_t, dstr_t, srcc_t, dstc_t, mbuf_v, rows_v, ones_v,
                  stage_v, stagec_v, acc_sh, cnt_sh,
                  sem_g0, sem_g1, sem_s0, sem_s1, sem_c):
    c = lax.axis_index("c")
    s = lax.axis_index("s")
    sem_g = (sem_g0, sem_g1)
    sem_s = (sem_s0, sem_s1)
    qbase = (c * NQ, (NC + c) * NQ)

    pltpu.sync_copy(ones_hbm, ones_v)
    pltpu.sync_copy(src_hbm.at[pl.ds(s * NCH, NCH)], srcr_t)
    pltpu.sync_copy(dst_hbm.at[pl.ds(s * NCH, NCH)], dstr_t)

    def _compact_row(j, carry):
        m0v, m1v = carry
        for i in range(B // 16):
            sl = pl.ds(i * 16, 16)
            dv = dstr_t[j, sl]
            sv = srcr_t[j, sl]
            t0 = dv - qbase[0]
            v0 = jnp.logical_and(t0 >= 0, t0 < NQ)
            pos0 = m0v - 1 + plsc.cumsum(jnp.where(v0, 1, 0))
            plsc.store_scatter(dstc_t, [pos0], t0, mask=v0)
            plsc.store_scatter(srcc_t, [pos0], sv, mask=v0)
            m0v = m0v + plsc.all_reduce_population_count(v0)
            t1 = dv - qbase[1]
            v1 = jnp.logical_and(t1 >= 0, t1 < NQ)
            pos1 = CAP - m1v - plsc.cumsum(jnp.where(v1, 1, 0))
            plsc.store_scatter(dstc_t, [pos1], t1, mask=v1)
            plsc.store_scatter(srcc_t, [pos1], sv, mask=v1)
            m1v = m1v + plsc.all_reduce_population_count(v1)
        return (m0v, m1v)

    z16 = jnp.zeros((16,), jnp.int32)
    m0v, m1v = lax.fori_loop(0, NCH, _compact_row, (z16, z16))
    mbuf_v[...] = m0v
    m0 = mbuf_v[pl.ds(0, 16)][0]
    mbuf_v[...] = m1v
    m1 = mbuf_v[pl.ds(0, 16)][0]

    mpad0 = ((m0 + 2 * B - 1) // (2 * B)) * (2 * B)
    mpad1 = ((m1 + 2 * B - 1) // (2 * B)) * (2 * B)
    grow16 = z16 + GROW

    def _pad(g, carry):
        off = g * 16 + lax.iota(jnp.int32, 16)
        idx0 = m0 + off
        mask0 = idx0 < mpad0
        plsc.store_scatter(dstc_t, [idx0], grow16, mask=mask0)
        plsc.store_scatter(srcc_t, [idx0], z16, mask=mask0)
        idx1 = CAP - m1 - 1 - off
        mask1 = off < (mpad1 - m1)
        plsc.store_scatter(dstc_t, [idx1], grow16, mask=mask1)
        plsc.store_scatter(srcc_t, [idx1], z16, mask=mask1)
        return carry

    lax.fori_loop(0, 2 * B // 16, _pad, 0)
    npairs = (mpad0 // (2 * B), mpad1 // (2 * B))

    def _idx(q, j):
        if q == 0:
            return pl.ds(j * B, B)
        return pl.ds(CAP - (j + 1) * B, B)

    def _gather(q, j, d):
        pltpu.async_copy(hq_hbm.at[srcc_t.at[_idx(q, j)]], rows_v.at[d],
                         sem_g[d])

    def _gather_wait(q, j, d):
        pltpu.make_async_copy(hq_hbm.at[srcc_t.at[_idx(q, j)]],
                              rows_v.at[d], sem_g[d]).wait()

    def _scat(q, j, d):
        pltpu.async_copy(rows_v.at[d], acc_sh.at[dstc_t.at[_idx(q, j)]],
                         sem_s[d], add=True)

    def _scat_wait(q, j, d):
        pltpu.make_async_copy(rows_v.at[d],
                              acc_sh.at[dstc_t.at[_idx(q, j)]],
                              sem_s[d]).wait()

    def _cnt(q, j):
        pltpu.async_copy(ones_v, cnt_sh.at[dstc_t.at[_idx(q, j)]],
                         sem_c, add=True)

    def _cnt_wait(q, j):
        pltpu.make_async_copy(ones_v, cnt_sh.at[dstc_t.at[_idx(q, j)]],
                              sem_c).wait()

    for p in range(NPH):
        if p > 0:
            def _bump(g, carry):
                sl = pl.ds(g * 16, 16)
                srcc_t[sl] = srcc_t[sl] + N
                return carry

            lax.fori_loop(0, CAP // 16, _bump, 0)

        for q in range(2):
            npair = npairs[q]
            pltpu.sync_copy(zrows_hbm, stage_v)
            pltpu.sync_copy(stage_v, acc_sh.at[pl.ds(s * SLAB, SLAB)])

            @pl.when(s == 0)
            def _():
                pltpu.sync_copy(stage_v.at[pl.ds(0, TAIL)],
                                acc_sh.at[pl.ds(TAIL_OFF, TAIL)])

            if p == 0:
                pltpu.sync_copy(zcnt_hbm, stagec_v)
                pltpu.sync_copy(stagec_v, cnt_sh.at[pl.ds(s * SLAB, SLAB)])

                @pl.when(s == 0)
                def _():
                    pltpu.sync_copy(stagec_v.at[pl.ds(0, TAIL)],
                                    cnt_sh.at[pl.ds(TAIL_OFF, TAIL)])

            plsc.subcore_barrier()

            @pl.when(npair > 0)
            def _():
                _gather(q, 0, 0)

            def _pipe(k, carry):
                a = 2 * k
                b = a + 1
                _gather_wait(q, a, 0)
                _scat(q, a, 0)

                @pl.when(k > 0)
                def _():
                    _scat_wait(q, a - 1, 1)

                _gather(q, b, 1)
                if p == 0:
                    @pl.when(k > 0)
                    def _():
                        _cnt_wait(q, a - 2)
                        _cnt_wait(q, a - 1)

                    _cnt(q, a)
                    _cnt(q, b)
                _gather_wait(q, b, 1)
                _scat(q, b, 1)

                @pl.when(k < npair - 1)
                def _():
                    _scat_wait(q, a, 0)
                    _gather(q, a + 2, 0)

                return carry

            lax.fori_loop(0, npair, _pipe, 0)

            @pl.when(npair > 0)
            def _():
                _scat_wait(q, 2 * npair - 2, 0)
                _scat_wait(q, 2 * npair - 1, 1)
                if p == 0:
                    _cnt_wait(q, 2 * npair - 2)
                    _cnt_wait(q, 2 * npair - 1)

            plsc.subcore_barrier()

            out0 = p * N + qbase[q]
            pltpu.sync_copy(acc_sh.at[pl.ds(s * SLAB, SLAB)], stage_v)
            pltpu.sync_copy(stage_v, agg_out.at[pl.ds(out0 + s * SLAB, SLAB)])

            @pl.when(s == 0)
            def _():
                pltpu.sync_copy(acc_sh.at[pl.ds(TAIL_OFF, TAIL)],
                                stage_v.at[pl.ds(0, TAIL)])
                pltpu.sync_copy(stage_v.at[pl.ds(0, TAIL)],
                                agg_out.at[pl.ds(out0 + TAIL_OFF, TAIL)])

            if p == 0:
                cb = qbase[q]
                pltpu.sync_copy(cnt_sh.at[pl.ds(s * SLAB, SLAB)], stagec_v)
                pltpu.sync_copy(stagec_v,
                                cnt_out.at[pl.ds(cb + s * SLAB, SLAB)])

                @pl.when(s == 0)
                def _():
                    pltpu.sync_copy(cnt_sh.at[pl.ds(TAIL_OFF, TAIL)],
                                    stagec_v.at[pl.ds(0, TAIL)])
                    pltpu.sync_copy(stagec_v.at[pl.ds(0, TAIL)],
                                    cnt_out.at[pl.ds(cb + TAIL_OFF, TAIL)])


def _out_body(*refs):
    agg_refs = refs[:NPH]
    c_ref = refs[NPH]
    h_refs = refs[NPH + 1:2 * NPH + 1]
    wl_ref, bl_ref, wr_ref, o_ref = refs[2 * NPH + 1:]
    inv = 1.0 / jnp.maximum(c_ref[:, 0:1], 1.0)
    dn = (((1,), (1,)), ((), ()))
    acc = bl_ref[...] + jnp.zeros((RB, D), jnp.float32)
    for q in range(NPH):
        acc += lax.dot_general(agg_refs[q][...] * inv,
                               wl_ref[:, q * DQ:(q + 1) * DQ], dn,
                               preferred_element_type=jnp.float32)
        acc += lax.dot_general(h_refs[q][...],
                               wr_ref[:, q * DQ:(q + 1) * DQ], dn,
                               preferred_element_type=jnp.float32)
    o_ref[...] = acc


def _group_spec(q):
    return pl.BlockSpec((RB, DQ), lambda i, q=q: (q * NRB + i, 0))


_out_call = pl.pallas_call(
    _out_body,
    grid=(NRB,),
    in_specs=(
        [_group_spec(q) for q in range(NPH)]
        + [pl.BlockSpec((RB, 16), lambda i: (i, 0))]
        + [_group_spec(q) for q in range(NPH)]
        + [
            pl.BlockSpec((D, D), lambda i: (0, 0)),
            pl.BlockSpec((1, D), lambda i: (0, 0)),
            pl.BlockSpec((D, D), lambda i: (0, 0)),
        ]
    ),
    out_specs=pl.BlockSpec((RB, D), lambda i: (i, 0)),
    out_shape=jax.ShapeDtypeStruct((N, D), jnp.float32),
)


def kernel(x, edge_index, gamma, beta, W_l, b_l, W_r):
    hq = _ln_call(x, gamma.reshape(1, D), beta.reshape(1, D))
    hqf = hq.reshape(NPH * N, DQ)
    src = jnp.concatenate(
        [edge_index[0], jnp.zeros((EPAD,), jnp.int32)]).reshape(EROWS, B)
    dst = jnp.concatenate(
        [edge_index[1], jnp.full((EPAD,), N, jnp.int32)]).reshape(EROWS, B)
    zrows = jnp.zeros((SLAB, DQ), jnp.float32)
    zcnt = jnp.zeros((SLAB, 16), jnp.float32)
    ones = jnp.ones((B, 16), jnp.float32)
    agg, cnt = _sc_aggregate(src, dst, hqf, zrows, zcnt, ones)
    args = ([agg] * NPH) + [cnt] + ([hqf] * NPH)
    return _out_call(*args, W_l, b_l.reshape(1, D), W_r)


# speedup vs baseline: 1.4721x; 1.2588x over previous
import functools

import jax
import jax.numpy as jnp
from jax import lax
from jax.experimental import pallas as pl
from jax.experimental.pallas import tpu as pltpu
from jax.experimental.pallas import tpu_sc as plsc

N = 10000
E = 160000
D = 256
EPS = 1e-5

NPH = 4
DQ = D // NPH

NC = 2
NS = 16
B = 128
NCH = 80
EROWS = NS * NCH
EPAD = EROWS * B - E
CAP = NCH * B + 2 * B
NHALF = N // NC
GROW = NHALF
ACCR = NHALF + 8
SLAB = 312
TAIL = NHALF - NS * SLAB
TAIL_OFF = NS * SLAB

RB = 1000
NRB = N // RB


def _ln_body(x_ref, g_ref, b_ref, o_ref):
    xb = x_ref[...]
    mu = jnp.mean(xb, axis=-1, keepdims=True)
    xc = xb - mu
    var = jnp.mean(xc * xc, axis=-1, keepdims=True)
    xn = xc * lax.rsqrt(var + EPS) * g_ref[...] + b_ref[...]
    h = jnp.maximum(xn, 0.0)
    for q in range(NPH):
        o_ref[q] = h[:, q * DQ:(q + 1) * DQ]


_ln_call = pl.pallas_call(
    _ln_body,
    grid=(NRB,),
    in_specs=[
        pl.BlockSpec((RB, D), lambda i: (i, 0)),
        pl.BlockSpec((1, D), lambda i: (0, 0)),
        pl.BlockSpec((1, D), lambda i: (0, 0)),
    ],
    out_specs=pl.BlockSpec((NPH, RB, DQ), lambda i: (0, i, 0)),
    out_shape=jax.ShapeDtypeStruct((NPH, N, DQ), jnp.float32),
)


_mesh = plsc.VectorSubcoreMesh(
    core_axis_name="c", subcore_axis_name="s", num_cores=NC, num_subcores=NS
)


@functools.partial(
    pl.kernel,
    out_type=(
        jax.ShapeDtypeStruct((NPH * N, DQ), jnp.float32),
        jax.ShapeDtypeStruct((N, 16), jnp.float32),
    ),
    mesh=_mesh,
    compiler_params=pltpu.CompilerParams(use_tc_tiling_on_sc=False,
                                         needs_layout_passes=False),
    scratch_types=(
        pltpu.VMEM((NCH, B), jnp.int32),
        pltpu.VMEM((NCH, B), jnp.int32),
        pltpu.VMEM((CAP,), jnp.int32),
        pltpu.VMEM((CAP,), jnp.int32),
        pltpu.VMEM((16,), jnp.int32),
        pltpu.VMEM((2, B, DQ), jnp.float32),
        pltpu.VMEM((B, 16), jnp.float32),
        pltpu.VMEM((SLAB, DQ), jnp.float32),
        pltpu.VMEM((SLAB, 16), jnp.float32),
        pltpu.VMEM_SHARED((ACCR, DQ), jnp.float32),
        pltpu.VMEM_SHARED((ACCR, 16), jnp.float32),
        pltpu.SemaphoreType.DMA,
        pltpu.SemaphoreType.DMA,
        pltpu.SemaphoreType.DMA,
        pltpu.SemaphoreType.DMA,
        pltpu.SemaphoreType.DMA,
    ),
)
def _sc_aggregate(src_hbm, dst_hbm, hq_hbm, zrows_hbm, zcnt_hbm, ones_hbm,
                  agg_out, cnt_out,
                  srcr_t, dstr_t, srcc_t, dstc_t, mbuf_v, rows_v, ones_v,
                  stage_v, stagec_v, acc_sh, cnt_sh,
                  sem_g0, sem_g1, sem_s0, sem_s1, sem_c):
    c = lax.axis_index("c")
    s = lax.axis_index("s")
    cbase = c * NHALF
    sem_g = (sem_g0, sem_g1)
    sem_s = (sem_s0, sem_s1)

    pltpu.sync_copy(ones_hbm, ones_v)
    pltpu.sync_copy(src_hbm.at[pl.ds(s * NCH, NCH)], srcr_t)
    pltpu.sync_copy(dst_hbm.at[pl.ds(s * NCH, NCH)], dstr_t)

    def _compact_row(j, mvec):
        for i in range(B // 16):
            sl = pl.ds(i * 16, 16)
            t = dstr_t[j, sl] - cbase
            valid = jnp.logical_and(t >= 0, t < NHALF)
            pos = mvec - 1 + plsc.cumsum(jnp.where(valid, 1, 0))
            plsc.store_scatter(dstc_t, [pos], t, mask=valid)
            plsc.store_scatter(srcc_t, [pos], srcr_t[j, sl], mask=valid)
            mvec = mvec + plsc.all_reduce_population_count(valid)
        return mvec

    mvec = lax.fori_loop(0, NCH, _compact_row,
                         jnp.zeros((16,), jnp.int32))
    mbuf_v[...] = mvec
    m = mbuf_v[pl.ds(0, 16)][0]

    mpad = ((m + 2 * B - 1) // (2 * B)) * (2 * B)
    zeros16 = jnp.zeros((16,), jnp.int32)
    grow16 = zeros16 + GROW

    def _pad(g, carry):
        idx = m + g * 16 + lax.iota(jnp.int32, 16)
        mask = idx < mpad
        plsc.store_scatter(dstc_t, [idx], grow16, mask=mask)
        plsc.store_scatter(srcc_t, [idx], zeros16, mask=mask)
        return carry

    lax.fori_loop(0, 2 * B // 16, _pad, 0)
    npair = mpad // (2 * B)

    def _gather(j, d):
        return pltpu.async_copy(hq_hbm.at[srcc_t.at[pl.ds(j * B, B)]],
                                rows_v.at[d], sem_g[d])

    def _gather_wait(j, d):
        pltpu.make_async_copy(hq_hbm.at[srcc_t.at[pl.ds(j * B, B)]],
                              rows_v.at[d], sem_g[d]).wait()

    def _scat(j, d):
        pltpu.async_copy(rows_v.at[d], acc_sh.at[dstc_t.at[pl.ds(j * B, B)]],
                         sem_s[d], add=True)

    def _scat_wait(j, d):
        pltpu.make_async_copy(rows_v.at[d],
                              acc_sh.at[dstc_t.at[pl.ds(j * B, B)]],
                              sem_s[d]).wait()

    def _cnt(j):
        pltpu.async_copy(ones_v, cnt_sh.at[dstc_t.at[pl.ds(j * B, B)]],
                         sem_c, add=True)

    def _cnt_wait(j):
        pltpu.make_async_copy(ones_v, cnt_sh.at[dstc_t.at[pl.ds(j * B, B)]],
                              sem_c).wait()

    for p in range(NPH):
        if p > 0:
            def _bump(g, carry):
                sl = pl.ds(g * 16, 16)
                srcc_t[sl] = srcc_t[sl] + N
                return carry

            lax.fori_loop(0, CAP // 16, _bump, 0)

        pltpu.sync_copy(zrows_hbm, stage_v)
        pltpu.sync_copy(stage_v, acc_sh.at[pl.ds(s * SLAB, SLAB)])

        @pl.when(s == 0)
        def _():
            pltpu.sync_copy(stage_v.at[pl.ds(0, TAIL)],
                            acc_sh.at[pl.ds(TAIL_OFF, TAIL)])

        if p == 0:
            pltpu.sync_copy(zcnt_hbm, stagec_v)
            pltpu.sync_copy(stagec_v, cnt_sh.at[pl.ds(s * SLAB, SLAB)])

            @pl.when(s == 0)
            def _():
                pltpu.sync_copy(stagec_v.at[pl.ds(0, TAIL)],
                                cnt_sh.at[pl.ds(TAIL_OFF, TAIL)])

        plsc.subcore_barrier()

        @pl.when(npair > 0)
        def _():
            _gather(0, 0)

        def _pipe(k, carry):
            a = 2 * k
            b = a + 1
            _gather_wait(a, 0)
            _scat(a, 0)

            @pl.when(k > 0)
            def _():
                _scat_wait(a - 1, 1)

            _gather(b, 1)
            if p == 0:
                @pl.when(k > 0)
                def _():
                    _cnt_wait(a - 2)
                    _cnt_wait(a - 1)

                _cnt(a)
                _cnt(b)
            _gather_wait(b, 1)
            _scat(b, 1)

            @pl.when(k < npair - 1)
            def _():
                _scat_wait(a, 0)
                _gather(a + 2, 0)

            return carry

        lax.fori_loop(0, npair, _pipe, 0)

        @pl.when(npair > 0)
        def _():
            _scat_wait(2 * npair - 2, 0)
            _scat_wait(2 * npair - 1, 1)
            if p == 0:
                _cnt_wait(2 * npair - 2)
                _cnt_wait(2 * npair - 1)

        plsc.subcore_barrier()

        out0 = p * N + cbase
        pltpu.sync_copy(acc_sh.at[pl.ds(s * SLAB, SLAB)], stage_v)
        pltpu.sync_copy(stage_v, agg_out.at[pl.ds(out0 + s * SLAB, SLAB)])

        @pl.when(s == 0)
        def _():
            pltpu.sync_copy(acc_sh.at[pl.ds(TAIL_OFF, TAIL)],
                            stage_v.at[pl.ds(0, TAIL)])
            pltpu.sync_copy(stage_v.at[pl.ds(0, TAIL)],
                            agg_out.at[pl.ds(out0 + TAIL_OFF, TAIL)])

        if p == 0:
            pltpu.sync_copy(cnt_sh.at[pl.ds(s * SLAB, SLAB)], stagec_v)
            pltpu.sync_copy(stagec_v, cnt_out.at[pl.ds(cbase + s * SLAB, SLAB)])

            @pl.when(s == 0)
            def _():
                pltpu.sync_copy(cnt_sh.at[pl.ds(TAIL_OFF, TAIL)],
                                stagec_v.at[pl.ds(0, TAIL)])
                pltpu.sync_copy(stagec_v.at[pl.ds(0, TAIL)],
                                cnt_out.at[pl.ds(cbase + TAIL_OFF, TAIL)])


def _out_body(*refs):
    agg_refs = refs[:NPH]
    c_ref = refs[NPH]
    h_refs = refs[NPH + 1:2 * NPH + 1]
    wl_ref, bl_ref, wr_ref, o_ref = refs[2 * NPH + 1:]
    inv = 1.0 / jnp.maximum(c_ref[:, 0:1], 1.0)
    dn = (((1,), (1,)), ((), ()))
    acc = bl_ref[...] + jnp.zeros((RB, D), jnp.float32)
    for q in range(NPH):
        acc += lax.dot_general(agg_refs[q][...] * inv,
                               wl_ref[:, q * DQ:(q + 1) * DQ], dn,
                               preferred_element_type=jnp.float32)
        acc += lax.dot_general(h_refs[q][...],
                               wr_ref[:, q * DQ:(q + 1) * DQ], dn,
                               preferred_element_type=jnp.float32)
    o_ref[...] = acc


def _group_spec(q):
    return pl.BlockSpec((RB, DQ), lambda i, q=q: (q * NRB + i, 0))


_out_call = pl.pallas_call(
    _out_body,
    grid=(NRB,),
    in_specs=(
        [_group_spec(q) for q in range(NPH)]
        + [pl.BlockSpec((RB, 16), lambda i: (i, 0))]
        + [_group_spec(q) for q in range(NPH)]
        + [
            pl.BlockSpec((D, D), lambda i: (0, 0)),
            pl.BlockSpec((1, D), lambda i: (0, 0)),
            pl.BlockSpec((D, D), lambda i: (0, 0)),
        ]
    ),
    out_specs=pl.BlockSpec((RB, D), lambda i: (i, 0)),
    out_shape=jax.ShapeDtypeStruct((N, D), jnp.float32),
)


def kernel(x, edge_index, gamma, beta, W_l, b_l, W_r):
    hq = _ln_call(x, gamma.reshape(1, D), beta.reshape(1, D))
    hqf = hq.reshape(NPH * N, DQ)
    src = jnp.concatenate(
        [edge_index[0], jnp.zeros((EPAD,), jnp.int32)]).reshape(EROWS, B)
    dst = jnp.concatenate(
        [edge_index[1], jnp.full((EPAD,), N, jnp.int32)]).reshape(EROWS, B)
    zrows = jnp.zeros((SLAB, DQ), jnp.float32)
    zcnt = jnp.zeros((SLAB, 16), jnp.float32)
    ones = jnp.ones((B, 16), jnp.float32)
    agg, cnt = _sc_aggregate(src, dst, hqf, zrows, zcnt, ones)
    args = ([agg] * NPH) + [cnt] + ([hqf] * NPH)
    return _out_call(*args, W_l, b_l.reshape(1, D), W_r)


# speedup vs baseline: 1.4908x; 1.0127x over previous
import functools

import jax
import jax.numpy as jnp
from jax import lax
from jax.experimental import pallas as pl
from jax.experimental.pallas import tpu as pltpu
from jax.experimental.pallas import tpu_sc as plsc

N = 10000
E = 160000
D = 256
EPS = 1e-5

NPH = 4
DQ = D // NPH

NC = 2
NS = 16
B = 128
NCH = 80
EROWS = NS * NCH
EPAD = EROWS * B - E
CAP = NCH * B + 2 * B
NHALF = N // NC
GROW = NHALF
ACCR = NHALF + 8
SLAB = 312
TAIL = NHALF - NS * SLAB
TAIL_OFF = NS * SLAB

RB = 1000
NRB = N // RB


def _ln_body(x_ref, g_ref, b_ref, o_ref):
    xb = x_ref[...]
    mu = jnp.mean(xb, axis=-1, keepdims=True)
    xc = xb - mu
    var = jnp.mean(xc * xc, axis=-1, keepdims=True)
    xn = xc * lax.rsqrt(var + EPS) * g_ref[...] + b_ref[...]
    h = jnp.maximum(xn, 0.0)
    for q in range(NPH):
        o_ref[q] = h[:, q * DQ:(q + 1) * DQ]


_ln_call = pl.pallas_call(
    _ln_body,
    grid=(NRB,),
    in_specs=[
        pl.BlockSpec((RB, D), lambda i: (i, 0)),
        pl.BlockSpec((1, D), lambda i: (0, 0)),
        pl.BlockSpec((1, D), lambda i: (0, 0)),
    ],
    out_specs=pl.BlockSpec((NPH, RB, DQ), lambda i: (0, i, 0)),
    out_shape=jax.ShapeDtypeStruct((NPH, N, DQ), jnp.float32),
)


_mesh = plsc.VectorSubcoreMesh(
    core_axis_name="c", subcore_axis_name="s", num_cores=NC, num_subcores=NS
)


@functools.partial(
    pl.kernel,
    out_type=(
        jax.ShapeDtypeStruct((NPH * N, DQ), jnp.float32),
        jax.ShapeDtypeStruct((N, 16), jnp.float32),
    ),
    mesh=_mesh,
    compiler_params=pltpu.CompilerParams(use_tc_tiling_on_sc=False,
                                         needs_layout_passes=False),
    scratch_types=(
        pltpu.VMEM((NCH, B), jnp.int32),
        pltpu.VMEM((NCH, B), jnp.int32),
        pltpu.VMEM((CAP,), jnp.int32),
        pltpu.VMEM((CAP,), jnp.int32),
        pltpu.VMEM((16,), jnp.int32),
        pltpu.VMEM((2, B, DQ), jnp.float32),
        pltpu.VMEM((B, 16), jnp.float32),
        pltpu.VMEM((SLAB, DQ), jnp.float32),
        pltpu.VMEM((SLAB, DQ), jnp.float32),
        pltpu.VMEM((SLAB, 16), jnp.float32),
        pltpu.VMEM_SHARED((ACCR, DQ), jnp.float32),
        pltpu.VMEM_SHARED((ACCR, 16), jnp.float32),
        pltpu.SemaphoreType.DMA,
        pltpu.SemaphoreType.DMA,
        pltpu.SemaphoreType.DMA,
        pltpu.SemaphoreType.DMA,
        pltpu.SemaphoreType.DMA,
    ),
)
def _sc_aggregate(src_hbm, dst_hbm, hq_hbm, zrows_hbm, zcnt_hbm, ones_hbm,
                  agg_out, cnt_out,
                  srcr_t, dstr_t, srcc_t, dstc_t, mbuf_v, rows_v, ones_v,
                  stage_v, zbuf_v, stagec_v, acc_sh, cnt_sh,
                  sem_g0, sem_g1, sem_s0, sem_s1, sem_c):
    c = lax.axis_index("c")
    s = lax.axis_index("s")
    cbase = c * NHALF
    sem_g = (sem_g0, sem_g1)
    sem_s = (sem_s0, sem_s1)

    pltpu.sync_copy(ones_hbm, ones_v)
    pltpu.sync_copy(src_hbm.at[pl.ds(s * NCH, NCH)], srcr_t)
    pltpu.sync_copy(dst_hbm.at[pl.ds(s * NCH, NCH)], dstr_t)

    def _compact_row(j, mvec):
        for i in range(B // 16):
            sl = pl.ds(i * 16, 16)
            t = dstr_t[j, sl] - cbase
            valid = jnp.logical_and(t >= 0, t < NHALF)
            pos = mvec - 1 + plsc.cumsum(jnp.where(valid, 1, 0))
            plsc.store_scatter(dstc_t, [pos], t, mask=valid)
            plsc.store_scatter(srcc_t, [pos], srcr_t[j, sl], mask=valid)
            mvec = mvec + plsc.all_reduce_population_count(valid)
        return mvec

    mvec = lax.fori_loop(0, NCH, _compact_row,
                         jnp.zeros((16,), jnp.int32))
    mbuf_v[...] = mvec
    m = mbuf_v[pl.ds(0, 16)][0]

    mpad = ((m + 2 * B - 1) // (2 * B)) * (2 * B)
    zeros16 = jnp.zeros((16,), jnp.int32)
    grow16 = zeros16 + GROW

    def _pad(g, carry):
        idx = m + g * 16 + lax.iota(jnp.int32, 16)
        mask = idx < mpad
        plsc.store_scatter(dstc_t, [idx], grow16, mask=mask)
        plsc.store_scatter(srcc_t, [idx], zeros16, mask=mask)
        return carry

    lax.fori_loop(0, 2 * B // 16, _pad, 0)
    npair = mpad // (2 * B)

    def _gather(j, d):
        return pltpu.async_copy(hq_hbm.at[srcc_t.at[pl.ds(j * B, B)]],
                                rows_v.at[d], sem_g[d])

    def _gather_wait(j, d):
        pltpu.make_async_copy(hq_hbm.at[srcc_t.at[pl.ds(j * B, B)]],
                              rows_v.at[d], sem_g[d]).wait()

    def _scat(j, d):
        pltpu.async_copy(rows_v.at[d], acc_sh.at[dstc_t.at[pl.ds(j * B, B)]],
                         sem_s[d], add=True)

    def _scat_wait(j, d):
        pltpu.make_async_copy(rows_v.at[d],
                              acc_sh.at[dstc_t.at[pl.ds(j * B, B)]],
                              sem_s[d]).wait()

    def _cnt(j):
        pltpu.async_copy(ones_v, cnt_sh.at[dstc_t.at[pl.ds(j * B, B)]],
                         sem_c, add=True)

    def _cnt_wait(j):
        pltpu.make_async_copy(ones_v, cnt_sh.at[dstc_t.at[pl.ds(j * B, B)]],
                              sem_c).wait()

    for p in range(NPH):
        if p > 0:
            def _bump(g, carry):
                sl = pl.ds(g * 16, 16)
                srcc_t[sl] = srcc_t[sl] + N
                return carry

            lax.fori_loop(0, CAP // 16, _bump, 0)

        if p == 0:
            pltpu.sync_copy(zrows_hbm, zbuf_v)
        pltpu.sync_copy(zbuf_v, acc_sh.at[pl.ds(s * SLAB, SLAB)])

        @pl.when(s == 0)
        def _():
            pltpu.sync_copy(zbuf_v.at[pl.ds(0, TAIL)],
                            acc_sh.at[pl.ds(TAIL_OFF, TAIL)])

        if p == 0:
            pltpu.sync_copy(zcnt_hbm, stagec_v)
            pltpu.sync_copy(stagec_v, cnt_sh.at[pl.ds(s * SLAB, SLAB)])

            @pl.when(s == 0)
            def _():
                pltpu.sync_copy(stagec_v.at[pl.ds(0, TAIL)],
                                cnt_sh.at[pl.ds(TAIL_OFF, TAIL)])

        plsc.subcore_barrier()

        @pl.when(npair > 0)
        def _():
            _gather(0, 0)

        def _pipe(k, carry):
            a = 2 * k
            b = a + 1
            _gather_wait(a, 0)
            _scat(a, 0)

            @pl.when(k > 0)
            def _():
                _scat_wait(a - 1, 1)

            _gather(b, 1)
            if p == 0:
                @pl.when(k > 3)
                def _():
                    _cnt_wait(a - 8)
                    _cnt_wait(a - 7)

                _cnt(a)
                _cnt(b)
            _gather_wait(b, 1)
            _scat(b, 1)

            @pl.when(k < npair - 1)
            def _():
                _scat_wait(a, 0)
                _gather(a + 2, 0)

            return carry

        lax.fori_loop(0, npair, _pipe, 0)

        @pl.when(npair > 0)
        def _():
            _scat_wait(2 * npair - 2, 0)
            _scat_wait(2 * npair - 1, 1)

        if p == 0:
            def _drain(j, carry):
                _cnt_wait(j)
                return carry

            lax.fori_loop(jnp.maximum(2 * npair - 8, 0), 2 * npair,
                          _drain, 0)

        plsc.subcore_barrier()

        out0 = p * N + cbase
        pltpu.sync_copy(acc_sh.at[pl.ds(s * SLAB, SLAB)], stage_v)
        pltpu.sync_copy(stage_v, agg_out.at[pl.ds(out0 + s * SLAB, SLAB)])

        @pl.when(s == 0)
        def _():
            pltpu.sync_copy(acc_sh.at[pl.ds(TAIL_OFF, TAIL)],
                            stage_v.at[pl.ds(0, TAIL)])
            pltpu.sync_copy(stage_v.at[pl.ds(0, TAIL)],
                            agg_out.at[pl.ds(out0 + TAIL_OFF, TAIL)])

        if p == 0:
            pltpu.sync_copy(cnt_sh.at[pl.ds(s * SLAB, SLAB)], stagec_v)
            pltpu.sync_copy(stagec_v, cnt_out.at[pl.ds(cbase + s * SLAB, SLAB)])

            @pl.when(s == 0)
            def _():
                pltpu.sync_copy(cnt_sh.at[pl.ds(TAIL_OFF, TAIL)],
                                stagec_v.at[pl.ds(0, TAIL)])
                pltpu.sync_copy(stagec_v.at[pl.ds(0, TAIL)],
                                cnt_out.at[pl.ds(cbase + TAIL_OFF, TAIL)])


def _out_body(*refs):
    agg_refs = refs[:NPH]
    c_ref = refs[NPH]
    h_refs = refs[NPH + 1:2 * NPH + 1]
    wl_ref, bl_ref, wr_ref, o_ref = refs[2 * NPH + 1:]
    inv = 1.0 / jnp.maximum(c_ref[:, 0:1], 1.0)
    dn = (((1,), (1,)), ((), ()))
    acc = bl_ref[...] + jnp.zeros((RB, D), jnp.float32)
    for q in range(NPH):
        acc += lax.dot_general(agg_refs[q][...] * inv,
                               wl_ref[:, q * DQ:(q + 1) * DQ], dn,
                               preferred_element_type=jnp.float32)
        acc += lax.dot_general(h_refs[q][...],
                               wr_ref[:, q * DQ:(q + 1) * DQ], dn,
                               preferred_element_type=jnp.float32)
    o_ref[...] = acc


def _group_spec(q):
    return pl.BlockSpec((RB, DQ), lambda i, q=q: (q * NRB + i, 0))


_out_call = pl.pallas_call(
    _out_body,
    grid=(NRB,),
    in_specs=(
        [_group_spec(q) for q in range(NPH)]
        + [pl.BlockSpec((RB, 16), lambda i: (i, 0))]
        + [_group_spec(q) for q in range(NPH)]
        + [
            pl.BlockSpec((D, D), lambda i: (0, 0)),
            pl.BlockSpec((1, D), lambda i: (0, 0)),
            pl.BlockSpec((D, D), lambda i: (0, 0)),
        ]
    ),
    out_specs=pl.BlockSpec((RB, D), lambda i: (i, 0)),
    out_shape=jax.ShapeDtypeStruct((N, D), jnp.float32),
)


def kernel(x, edge_index, gamma, beta, W_l, b_l, W_r):
    hq = _ln_call(x, gamma.reshape(1, D), beta.reshape(1, D))
    hqf = hq.reshape(NPH * N, DQ)
    src = jnp.concatenate(
        [edge_index[0], jnp.zeros((EPAD,), jnp.int32)]).reshape(EROWS, B)
    dst = jnp.concatenate(
        [edge_index[1], jnp.full((EPAD,), N, jnp.int32)]).reshape(EROWS, B)
    zrows = jnp.zeros((SLAB, DQ), jnp.float32)
    zcnt = jnp.zeros((SLAB, 16), jnp.float32)
    ones = jnp.ones((B, 16), jnp.float32)
    agg, cnt = _sc_aggregate(src, dst, hqf, zrows, zcnt, ones)
    args = ([agg] * NPH) + [cnt] + ([hqf] * NPH)
    return _out_call(*args, W_l, b_l.reshape(1, D), W_r)


# speedup vs baseline: 1.5781x; 1.0586x over previous
import functools

import jax
import jax.numpy as jnp
from jax import lax
from jax.experimental import pallas as pl
from jax.experimental.pallas import tpu as pltpu
from jax.experimental.pallas import tpu_sc as plsc

N = 10000
E = 160000
D = 256
EPS = 1e-5

NPH = 4
DQ = D // NPH

NC = 2
NS = 16
B = 128
NCH = 80
EROWS = NS * NCH
EPAD = EROWS * B - E
CAP = NCH * B + 2 * B
NHALF = N // NC
GROW = NHALF
ACCR = NHALF + 8
SLAB = 312
TAIL = NHALF - NS * SLAB
TAIL_OFF = NS * SLAB

RB = 1000
NRB = N // RB


def _ln_body(x_ref, g_ref, b_ref, o_ref):
    xb = x_ref[...]
    mu = jnp.mean(xb, axis=-1, keepdims=True)
    xc = xb - mu
    var = jnp.mean(xc * xc, axis=-1, keepdims=True)
    xn = xc * lax.rsqrt(var + EPS) * g_ref[...] + b_ref[...]
    h = jnp.maximum(xn, 0.0)
    for q in range(NPH):
        o_ref[q] = h[:, q * DQ:(q + 1) * DQ]


_ln_call = pl.pallas_call(
    _ln_body,
    grid=(NRB,),
    in_specs=[
        pl.BlockSpec((RB, D), lambda i: (i, 0)),
        pl.BlockSpec((1, D), lambda i: (0, 0)),
        pl.BlockSpec((1, D), lambda i: (0, 0)),
    ],
    out_specs=pl.BlockSpec((NPH, RB, DQ), lambda i: (0, i, 0)),
    out_shape=jax.ShapeDtypeStruct((NPH, N, DQ), jnp.float32),
)


_mesh = plsc.VectorSubcoreMesh(
    core_axis_name="c", subcore_axis_name="s", num_cores=NC, num_subcores=NS
)


@functools.partial(
    pl.kernel,
    out_type=(
        jax.ShapeDtypeStruct((NPH * N, DQ), jnp.float32),
        jax.ShapeDtypeStruct((N, 16), jnp.float32),
    ),
    mesh=_mesh,
    compiler_params=pltpu.CompilerParams(use_tc_tiling_on_sc=False,
                                         needs_layout_passes=False),
    scratch_types=(
        pltpu.VMEM((NCH, B), jnp.int32),
        pltpu.VMEM((NCH, B), jnp.int32),
        pltpu.VMEM((CAP,), jnp.int32),
        pltpu.VMEM((CAP,), jnp.int32),
        pltpu.VMEM((16,), jnp.int32),
        pltpu.VMEM((2, B, DQ), jnp.float32),
        pltpu.VMEM((B, 16), jnp.float32),
        pltpu.VMEM((SLAB, DQ), jnp.float32),
        pltpu.VMEM((SLAB, DQ), jnp.float32),
        pltpu.VMEM((SLAB, 16), jnp.float32),
        pltpu.VMEM_SHARED((ACCR, DQ), jnp.float32),
        pltpu.VMEM_SHARED((ACCR, 16), jnp.float32),
        pltpu.SemaphoreType.DMA,
        pltpu.SemaphoreType.DMA,
        pltpu.SemaphoreType.DMA,
        pltpu.SemaphoreType.DMA,
        pltpu.SemaphoreType.DMA,
    ),
)
def _sc_aggregate(src_hbm, dst_hbm, hq_hbm, zrows_hbm, zcnt_hbm, ones_hbm,
                  agg_out, cnt_out,
                  srcr_t, dstr_t, srcc_t, dstc_t, mbuf_v, rows_v, ones_v,
                  stage_v, zbuf_v, stagec_v, acc_sh, cnt_sh,
                  sem_g0, sem_g1, sem_s0, sem_s1, sem_c):
    c = lax.axis_index("c")
    s = lax.axis_index("s")
    cbase = c * NHALF
    sem_g = (sem_g0, sem_g1)
    sem_s = (sem_s0, sem_s1)

    pltpu.sync_copy(ones_hbm, ones_v)
    pltpu.sync_copy(src_hbm.at[pl.ds(s * NCH, NCH)], srcr_t)
    pltpu.sync_copy(dst_hbm.at[pl.ds(s * NCH, NCH)], dstr_t)

    def _compact_row(j, mvec):
        for i in range(B // 16):
            sl = pl.ds(i * 16, 16)
            t = dstr_t[j, sl] - cbase
            valid = jnp.logical_and(t >= 0, t < NHALF)
            pos = mvec - 1 + plsc.cumsum(jnp.where(valid, 1, 0))
            plsc.store_scatter(dstc_t, [pos], t, mask=valid)
            plsc.store_scatter(srcc_t, [pos], srcr_t[j, sl], mask=valid)
            mvec = mvec + plsc.all_reduce_population_count(valid)
        return mvec

    mvec = lax.fori_loop(0, NCH, _compact_row,
                         jnp.zeros((16,), jnp.int32))
    mbuf_v[...] = mvec
    m = mbuf_v[pl.ds(0, 16)][0]

    mpad = ((m + 2 * B - 1) // (2 * B)) * (2 * B)
    zeros16 = jnp.zeros((16,), jnp.int32)
    grow16 = zeros16 + GROW

    def _pad(g, carry):
        idx = m + g * 16 + lax.iota(jnp.int32, 16)
        mask = idx < mpad
        plsc.store_scatter(dstc_t, [idx], grow16, mask=mask)
        plsc.store_scatter(srcc_t, [idx], zeros16, mask=mask)
        return carry

    lax.fori_loop(0, 2 * B // 16, _pad, 0)
    npair = mpad // (2 * B)

    def _gather(j, d):
        return pltpu.async_copy(hq_hbm.at[srcc_t.at[pl.ds(j * B, B)]],
                                rows_v.at[d], sem_g[d])

    def _gather_wait(j, d):
        pltpu.make_async_copy(hq_hbm.at[srcc_t.at[pl.ds(j * B, B)]],
                              rows_v.at[d], sem_g[d]).wait()

    def _scat(j, d):
        pltpu.async_copy(rows_v.at[d], acc_sh.at[dstc_t.at[pl.ds(j * B, B)]],
                         sem_s[d], add=True)

    def _scat_wait(j, d):
        pltpu.make_async_copy(rows_v.at[d],
                              acc_sh.at[dstc_t.at[pl.ds(j * B, B)]],
                              sem_s[d]).wait()

    def _cnt(j):
        pltpu.async_copy(ones_v, cnt_sh.at[dstc_t.at[pl.ds(j * B, B)]],
                         sem_c, add=True)

    def _cnt_wait(j):
        pltpu.make_async_copy(ones_v, cnt_sh.at[dstc_t.at[pl.ds(j * B, B)]],
                              sem_c).wait()

    for p in range(NPH):
        if p > 0:
            def _bump(g, carry):
                sl = pl.ds(g * 16, 16)
                srcc_t[sl] = srcc_t[sl] + N
                return carry

            lax.fori_loop(0, CAP // 16, _bump, 0)

        if p == 0:
            pltpu.sync_copy(zrows_hbm, zbuf_v)
        pltpu.sync_copy(zbuf_v, acc_sh.at[pl.ds(s * SLAB, SLAB)])

        @pl.when(s == 0)
        def _():
            pltpu.sync_copy(zbuf_v.at[pl.ds(0, TAIL)],
                            acc_sh.at[pl.ds(TAIL_OFF, TAIL)])

        if p == 0:
            pltpu.sync_copy(zcnt_hbm, stagec_v)
            pltpu.sync_copy(stagec_v, cnt_sh.at[pl.ds(s * SLAB, SLAB)])

            @pl.when(s == 0)
            def _():
                pltpu.sync_copy(stagec_v.at[pl.ds(0, TAIL)],
                                cnt_sh.at[pl.ds(TAIL_OFF, TAIL)])

        plsc.subcore_barrier()

        @pl.when(npair > 0)
        def _():
            _gather(0, 0)

        def _pipe(k, carry):
            a = 2 * k
            b = a + 1

            @pl.when(k > 0)
            def _():
                _scat_wait(a - 1, 1)

            _gather(b, 1)
            _gather_wait(a, 0)
            _scat(a, 0)
            if p == 0:
                @pl.when(k > 3)
                def _():
                    _cnt_wait(a - 8)
                    _cnt_wait(a - 7)

                _cnt(a)
                _cnt(b)

            @pl.when(k < npair - 1)
            def _():
                _scat_wait(a, 0)
                _gather(a + 2, 0)

            _gather_wait(b, 1)
            _scat(b, 1)
            return carry

        lax.fori_loop(0, npair, _pipe, 0)

        @pl.when(npair > 0)
        def _():
            _scat_wait(2 * npair - 2, 0)
            _scat_wait(2 * npair - 1, 1)

        if p == 0:
            def _drain(j, carry):
                _cnt_wait(j)
                return carry

            lax.fori_loop(jnp.maximum(2 * npair - 8, 0), 2 * npair,
                          _drain, 0)

        plsc.subcore_barrier()

        out0 = p * N + cbase
        pltpu.sync_copy(acc_sh.at[pl.ds(s * SLAB, SLAB)], stage_v)
        pltpu.sync_copy(stage_v, agg_out.at[pl.ds(out0 + s * SLAB, SLAB)])

        @pl.when(s == 0)
        def _():
            pltpu.sync_copy(acc_sh.at[pl.ds(TAIL_OFF, TAIL)],
                            stage_v.at[pl.ds(0, TAIL)])
            pltpu.sync_copy(stage_v.at[pl.ds(0, TAIL)],
                            agg_out.at[pl.ds(out0 + TAIL_OFF, TAIL)])

        if p == 0:
            pltpu.sync_copy(cnt_sh.at[pl.ds(s * SLAB, SLAB)], stagec_v)
            pltpu.sync_copy(stagec_v, cnt_out.at[pl.ds(cbase + s * SLAB, SLAB)])

            @pl.when(s == 0)
            def _():
                pltpu.sync_copy(cnt_sh.at[pl.ds(TAIL_OFF, TAIL)],
                                stagec_v.at[pl.ds(0, TAIL)])
                pltpu.sync_copy(stagec_v.at[pl.ds(0, TAIL)],
                                cnt_out.at[pl.ds(cbase + TAIL_OFF, TAIL)])


def _out_body(*refs):
    agg_refs = refs[:NPH]
    c_ref = refs[NPH]
    h_refs = refs[NPH + 1:2 * NPH + 1]
    wl_ref, bl_ref, wr_ref, o_ref = refs[2 * NPH + 1:]
    inv = 1.0 / jnp.maximum(c_ref[:, 0:1], 1.0)
    dn = (((1,), (1,)), ((), ()))
    acc = bl_ref[...] + jnp.zeros((RB, D), jnp.float32)
    for q in range(NPH):
        acc += lax.dot_general(agg_refs[q][...] * inv,
                               wl_ref[:, q * DQ:(q + 1) * DQ], dn,
                               preferred_element_type=jnp.float32)
        acc += lax.dot_general(h_refs[q][...],
                               wr_ref[:, q * DQ:(q + 1) * DQ], dn,
                               preferred_element_type=jnp.float32)
    o_ref[...] = acc


def _group_spec(q):
    return pl.BlockSpec((RB, DQ), lambda i, q=q: (q * NRB + i, 0))


_out_call = pl.pallas_call(
    _out_body,
    grid=(NRB,),
    in_specs=(
        [_group_spec(q) for q in range(NPH)]
        + [pl.BlockSpec((RB, 16), lambda i: (i, 0))]
        + [_group_spec(q) for q in range(NPH)]
        + [
            pl.BlockSpec((D, D), lambda i: (0, 0)),
            pl.BlockSpec((1, D), lambda i: (0, 0)),
            pl.BlockSpec((D, D), lambda i: (0, 0)),
        ]
    ),
    out_specs=pl.BlockSpec((RB, D), lambda i: (i, 0)),
    out_shape=jax.ShapeDtypeStruct((N, D), jnp.float32),
)


def kernel(x, edge_index, gamma, beta, W_l, b_l, W_r):
    hq = _ln_call(x, gamma.reshape(1, D), beta.reshape(1, D))
    hqf = hq.reshape(NPH * N, DQ)
    src = jnp.concatenate(
        [edge_index[0], jnp.zeros((EPAD,), jnp.int32)]).reshape(EROWS, B)
    dst = jnp.concatenate(
        [edge_index[1], jnp.full((EPAD,), N, jnp.int32)]).reshape(EROWS, B)
    zrows = jnp.zeros((SLAB, DQ), jnp.float32)
    zcnt = jnp.zeros((SLAB, 16), jnp.float32)
    ones = jnp.ones((B, 16), jnp.float32)
    agg, cnt = _sc_aggregate(src, dst, hqf, zrows, zcnt, ones)
    args = ([agg] * NPH) + [cnt] + ([hqf] * NPH)
    return _out_call(*args, W_l, b_l.reshape(1, D), W_r)
